# Initial kernel scaffold; baseline (speedup 1.0000x reference)
import jax
import jax.numpy as jnp
from jax.experimental import pallas as pl


def kernel(node_feats, adj_src, adj_dst, adj_vals, mask, Wm, bm, Wu, bu):
    raise NotImplementedError("write your pallas kernel here")


# speedup vs baseline: 2.7839x; 2.7839x over previous
import functools

import jax
import jax.numpy as jnp
from jax import lax
from jax.experimental import pallas as pl
from jax.experimental.pallas import tpu as pltpu
from jax.experimental.pallas import tpu_sc as plsc

N = 10000
E = 320000
D = 128
CH = 32
CHB = 64
NW = 32
NPAD = 10240
EPAD = 323584
EPW = EPAD // NW
NCHUNK = EPW // CH
NCHUNKB = EPW // CHB
NBLK = NPAD // CH
NBLKB = NPAD // CHB
JUNK = N + 16


def _pq_body(x_ref, wm_ref, bm_ref, p_ref, q_ref):
    x = x_ref[...]
    wm = wm_ref[...]
    p_ref[...] = jnp.dot(x, wm[:D], preferred_element_type=jnp.float32) + bm_ref[...]
    q_ref[...] = jnp.dot(x, wm[D:], preferred_element_type=jnp.float32)


def _pq_call(xpad, wm, bm2d):
    rb = NPAD // 8
    return pl.pallas_call(
        _pq_body,
        grid=(8,),
        in_specs=[
            pl.BlockSpec((rb, D), lambda i: (i, 0)),
            pl.BlockSpec((2 * D, D), lambda i: (0, 0)),
            pl.BlockSpec((1, D), lambda i: (0, 0)),
        ],
        out_specs=[
            pl.BlockSpec((rb, D), lambda i: (i, 0)),
            pl.BlockSpec((rb, D), lambda i: (i, 0)),
        ],
        out_shape=[
            jax.ShapeDtypeStruct((NPAD, D), jnp.float32),
            jax.ShapeDtypeStruct((NPAD, D), jnp.float32),
        ],
    )(xpad, wm, bm2d)


def _sc_body(p_hbm, q_hbm, src_hbm, dst_hbm, val_hbm,
             acc_out,
             sidx, didx, vals, prow, qrow,
             acc_sh, sem1, sem2):
    c = lax.axis_index("c")
    s = lax.axis_index("s")

    def _zrow(r, carry):
        for j in range(8):
            prow[r, pl.ds(j * 16, 16)] = jnp.zeros((16,), jnp.float32)
        return carry

    lax.fori_loop(0, CH, _zrow, 0)

    def _zblk(t, carry):
        b = s * (NBLK // 16) + t
        pltpu.sync_copy(prow, acc_sh.at[pl.ds(b * CH, CH)])
        return carry

    lax.fori_loop(0, NBLK // 16, _zblk, 0)
    plsc.subcore_barrier()

    base = (c * 16 + s) * EPW

    def _chunk(k, carry):
        off = base + k * CH
        pltpu.sync_copy(src_hbm.at[pl.ds(off, CH)], sidx)
        pltpu.sync_copy(dst_hbm.at[pl.ds(off, CH)], didx)
        pltpu.sync_copy(val_hbm.at[pl.ds(off, CH)], vals)
        cp1 = pltpu.async_copy(p_hbm.at[sidx], prow, sem1)
        cp2 = pltpu.async_copy(q_hbm.at[didx], qrow, sem2)
        cp1.wait()
        cp2.wait()

        def _grp(g, rc):
            v16 = vals[pl.ds(g * 16, 16)]
            for i in range(16):
                vb = v16[i]
                r = g * 16 + i
                for j in range(8):
                    pv = prow[r, pl.ds(j * 16, 16)]
                    qv = qrow[r, pl.ds(j * 16, 16)]
                    prow[r, pl.ds(j * 16, 16)] = jnp.maximum(pv + qv, 0.0) * vb
            return rc

        lax.fori_loop(0, CH // 16, _grp, 0)

        pltpu.sync_copy(prow, acc_sh.at[sidx], add=True)
        return carry

    lax.fori_loop(0, NCHUNK, _chunk, 0)
    plsc.subcore_barrier()

    def _out(t, carry):
        b = s * (NBLK // 16) + t
        pltpu.sync_copy(acc_sh.at[pl.ds(b * CH, CH)], prow)
        pltpu.sync_copy(prow, acc_out.at[c, pl.ds(b * CH, CH)])
        return carry

    lax.fori_loop(0, NBLK // 16, _out, 0)


_sc_call = functools.partial(
    pl.kernel,
    mesh=plsc.VectorSubcoreMesh(core_axis_name="c", subcore_axis_name="s"),
    out_type=jax.ShapeDtypeStruct((2, NPAD, D), jnp.float32),
    scratch_types=[
        pltpu.VMEM((CH,), jnp.int32),
        pltpu.VMEM((CH,), jnp.int32),
        pltpu.VMEM((CH,), jnp.float32),
        pltpu.VMEM((CH, D), jnp.float32),
        pltpu.VMEM((CH, D), jnp.float32),
        pltpu.VMEM_SHARED((NPAD, D), jnp.float32),
        pltpu.SemaphoreType.DMA,
        pltpu.SemaphoreType.DMA,
    ],
)(_sc_body)


def _cnt_body(src_hbm, cnt_out, sidx, ones, cnt_sh):
    c = lax.axis_index("c")
    s = lax.axis_index("s")

    def _zrow(r, carry):
        for j in range(8):
            ones[r, pl.ds(j * 16, 16)] = jnp.zeros((16,), jnp.float32)
        return carry

    lax.fori_loop(0, CHB, _zrow, 0)

    def _zblk(t, carry):
        b = s * (NBLKB // 16) + t
        pltpu.sync_copy(ones, cnt_sh.at[pl.ds(b * CHB, CHB)])
        return carry

    lax.fori_loop(0, NBLKB // 16, _zblk, 0)

    def _orow(r, carry):
        for j in range(8):
            ones[r, pl.ds(j * 16, 16)] = jnp.ones((16,), jnp.float32)
        return carry

    lax.fori_loop(0, CHB, _orow, 0)
    plsc.subcore_barrier()

    base = (c * 16 + s) * EPW

    def _chunk(k, carry):
        off = base + k * CHB
        pltpu.sync_copy(src_hbm.at[pl.ds(off, CHB)], sidx)

        pltpu.sync_copy(ones, cnt_sh.at[sidx], add=True)
        return carry

    lax.fori_loop(0, NCHUNKB, _chunk, 0)
    plsc.subcore_barrier()

    def _out(t, carry):
        b = s * (NBLKB // 16) + t
        pltpu.sync_copy(cnt_sh.at[pl.ds(b * CHB, CHB)], ones)
        pltpu.sync_copy(ones, cnt_out.at[c, pl.ds(b * CHB, CHB)])
        return carry

    lax.fori_loop(0, NBLKB // 16, _out, 0)


_cnt_call = functools.partial(
    pl.kernel,
    mesh=plsc.VectorSubcoreMesh(core_axis_name="c", subcore_axis_name="s"),
    out_type=jax.ShapeDtypeStruct((2, NPAD, D), jnp.float32),
    scratch_types=[
        pltpu.VMEM((CHB,), jnp.int32),
        pltpu.VMEM((CHB, D), jnp.float32),
        pltpu.VMEM_SHARED((NPAD, D), jnp.float32),
    ],
)(_cnt_body)


def _upd_body(x_ref, a0_ref, a1_ref, c0_ref, c1_ref, wu_ref, bu_ref, o_ref):
    x = x_ref[...]
    summed = a0_ref[...] + a1_ref[...]
    cnt = c0_ref[...][:, 0:1] + c1_ref[...][:, 0:1]
    g = jnp.where(cnt > 0, summed / jnp.maximum(cnt, 1.0), 0.0)
    wu = wu_ref[...]
    u = (jnp.dot(x, wu[:D], preferred_element_type=jnp.float32)
         + jnp.dot(g, wu[D:], preferred_element_type=jnp.float32)
         + bu_ref[...])
    o_ref[...] = jnp.maximum(u, 0.0)


def _upd_call(x, a0, a1, c0, c1, wu, bu2d):
    rb = 1000
    return pl.pallas_call(
        _upd_body,
        grid=(N // rb,),
        in_specs=[
            pl.BlockSpec((rb, D), lambda i: (i, 0)),
            pl.BlockSpec((rb, D), lambda i: (i, 0)),
            pl.BlockSpec((rb, D), lambda i: (i, 0)),
            pl.BlockSpec((rb, D), lambda i: (i, 0)),
            pl.BlockSpec((rb, D), lambda i: (i, 0)),
            pl.BlockSpec((2 * D, D), lambda i: (0, 0)),
            pl.BlockSpec((1, D), lambda i: (0, 0)),
        ],
        out_specs=pl.BlockSpec((rb, D), lambda i: (i, 0)),
        out_shape=jax.ShapeDtypeStruct((N, D), jnp.float32),
    )(x, a0, a1, c0, c1, wu, bu2d)


def kernel(node_feats, adj_src, adj_dst, adj_vals, mask, Wm, bm, Wu, bu):
    x = node_feats[0]
    xpad = jnp.concatenate([x, jnp.zeros((NPAD - N, D), jnp.float32)], axis=0)
    pad_src = JUNK + 16 + (jnp.arange(EPAD - E, dtype=jnp.int32) % 64)
    pad_dst = jnp.arange(EPAD - E, dtype=jnp.int32) % N
    srcp = jnp.concatenate([adj_src.astype(jnp.int32), pad_src])
    dstp = jnp.concatenate([adj_dst.astype(jnp.int32), pad_dst])
    valp = jnp.concatenate([adj_vals, jnp.zeros((EPAD - E,), jnp.float32)])

    p_tab, q_tab = _pq_call(xpad, Wm, bm.reshape(1, D))
    acc = _sc_call(p_tab, q_tab, srcp, dstp, valp)
    cnt = _cnt_call(srcp)
    out = _upd_call(x, acc[0, :N], acc[1, :N], cnt[0, :N], cnt[1, :N],
                    Wu, bu.reshape(1, D))
    return out[None]
